# Initial kernel scaffold; baseline (speedup 1.0000x reference)
import jax
import jax.numpy as jnp
from jax.experimental import pallas as pl


def kernel(x, knot_ys):
    raise NotImplementedError("write your pallas kernel here")


# speedup vs baseline: 7174.5658x; 7174.5658x over previous
import functools

import jax
import jax.numpy as jnp
import numpy as np
from jax.experimental import pallas as pl

_NUM_KNOTS = 10
_SPLINE_RANGE = 3.0
_KNOT_XS = np.linspace(-_SPLINE_RANGE, _SPLINE_RANGE, _NUM_KNOTS).astype(np.float32)


def _spline_body(ys_ref, x_ref, o_ref):
    ys = [ys_ref[0, j] for j in range(_NUM_KNOTS)]
    slopes = [
        (ys[j + 1] - ys[j]) / (float(_KNOT_XS[j + 1]) - float(_KNOT_XS[j]))
        for j in range(_NUM_KNOTS - 1)
    ]
    deltas = [slopes[0]] + [slopes[j] - slopes[j - 1] for j in range(1, _NUM_KNOTS - 1)]

    xc = jnp.clip(x_ref[...], -_SPLINE_RANGE, _SPLINE_RANGE)
    y = jnp.full_like(xc, 0.0) + ys[0]
    for j in range(_NUM_KNOTS - 1):
        y = y + deltas[j] * jnp.maximum(xc - float(_KNOT_XS[j]), 0.0)
    o_ref[...] = y


@functools.partial(jax.jit, static_argnames=())
def kernel(x, knot_ys):
    orig_shape = x.shape
    rows = x.shape[0] * x.shape[1]
    cols = x.shape[2]
    x2 = x.reshape(rows, cols)
    ys2 = knot_ys.reshape(1, _NUM_KNOTS)

    block_rows = 256
    grid = (rows // block_rows,)
    out = pl.pallas_call(
        _spline_body,
        grid=grid,
        in_specs=[
            pl.BlockSpec((1, _NUM_KNOTS), lambda i: (0, 0)),
            pl.BlockSpec((block_rows, cols), lambda i: (i, 0)),
        ],
        out_specs=pl.BlockSpec((block_rows, cols), lambda i: (i, 0)),
        out_shape=jax.ShapeDtypeStruct((rows, cols), x.dtype),
    )(ys2, x2)
    return out.reshape(orig_shape)


# speedup vs baseline: 17013.2285x; 2.3713x over previous
import functools

import jax
import jax.numpy as jnp
import numpy as np
from jax.experimental import pallas as pl

_NUM_KNOTS = 10
_SPLINE_RANGE = 3.0
_KNOT_XS = np.linspace(-_SPLINE_RANGE, _SPLINE_RANGE, _NUM_KNOTS).astype(np.float32)


_CHUNK_R = 8
_CHUNK_C = 2048


def _spline_body(ys_ref, x_ref, o_ref):
    ys = [ys_ref[0, j] for j in range(_NUM_KNOTS)]
    slopes = [
        (ys[j + 1] - ys[j]) / (float(_KNOT_XS[j + 1]) - float(_KNOT_XS[j]))
        for j in range(_NUM_KNOTS - 1)
    ]
    deltas = [slopes[0]] + [slopes[j] - slopes[j - 1] for j in range(1, _NUM_KNOTS - 1)]

    const = ys[0]
    for j in range(_NUM_KNOTS - 1):
        const = const - deltas[j] * float(_KNOT_XS[j])

    rows, cols = x_ref.shape
    ncol = cols // _CHUNK_C
    nchunks = (rows // _CHUNK_R) * ncol

    def chunk(i, carry):
        r = (i // ncol) * _CHUNK_R
        c = (i % ncol) * _CHUNK_C
        xv = x_ref[pl.ds(r, _CHUNK_R), pl.ds(c, _CHUNK_C)]
        xm = jnp.minimum(xv, _SPLINE_RANGE)
        terms = [deltas[j] * jnp.maximum(xm, float(_KNOT_XS[j]))
                 for j in range(_NUM_KNOTS - 1)]
        t01 = terms[0] + terms[1]
        t23 = terms[2] + terms[3]
        t45 = terms[4] + terms[5]
        t67 = terms[6] + terms[7]
        t8c = terms[8] + const
        y = (t01 + t23) + (t45 + t67) + t8c
        o_ref[pl.ds(r, _CHUNK_R), pl.ds(c, _CHUNK_C)] = y
        return carry

    jax.lax.fori_loop(0, nchunks, chunk, 0)


@functools.partial(jax.jit, static_argnames=())
def kernel(x, knot_ys):
    orig_shape = x.shape
    rows = x.shape[0] * x.shape[1]
    cols = x.shape[2]
    x2 = x.reshape(rows, cols)
    ys2 = knot_ys.reshape(1, _NUM_KNOTS)

    block_rows = 256
    grid = (rows // block_rows,)
    out = pl.pallas_call(
        _spline_body,
        grid=grid,
        in_specs=[
            pl.BlockSpec((1, _NUM_KNOTS), lambda i: (0, 0)),
            pl.BlockSpec((block_rows, cols), lambda i: (i, 0)),
        ],
        out_specs=pl.BlockSpec((block_rows, cols), lambda i: (i, 0)),
        out_shape=jax.ShapeDtypeStruct((rows, cols), x.dtype),
    )(ys2, x2)
    return out.reshape(orig_shape)


# speedup vs baseline: 17127.3995x; 1.0067x over previous
import functools

import jax
import jax.numpy as jnp
import numpy as np
from jax.experimental import pallas as pl

_NUM_KNOTS = 10
_SPLINE_RANGE = 3.0
_KNOT_XS = np.linspace(-_SPLINE_RANGE, _SPLINE_RANGE, _NUM_KNOTS).astype(np.float32)


_CHUNK_R = 8
_CHUNK_C = 2048


def _spline_body(ys_ref, x_ref, o_ref):
    ys = [ys_ref[0, j] for j in range(_NUM_KNOTS)]
    slopes = [
        (ys[j + 1] - ys[j]) / (float(_KNOT_XS[j + 1]) - float(_KNOT_XS[j]))
        for j in range(_NUM_KNOTS - 1)
    ]
    deltas = [slopes[0]] + [slopes[j] - slopes[j - 1] for j in range(1, _NUM_KNOTS - 1)]

    const = ys[0]
    for j in range(_NUM_KNOTS - 1):
        const = const - deltas[j] * float(_KNOT_XS[j])

    rows, cols = x_ref.shape
    ncol = cols // _CHUNK_C
    nchunks = (rows // _CHUNK_R) * ncol

    def chunk(i, carry):
        r = (i // ncol) * _CHUNK_R
        c = (i % ncol) * _CHUNK_C
        xv = x_ref[pl.ds(r, _CHUNK_R), pl.ds(c, _CHUNK_C)]
        xm = jnp.minimum(xv, _SPLINE_RANGE)
        terms = [deltas[j] * jnp.maximum(xm, float(_KNOT_XS[j]))
                 for j in range(_NUM_KNOTS - 1)]
        t01 = terms[0] + terms[1]
        t23 = terms[2] + terms[3]
        t45 = terms[4] + terms[5]
        t67 = terms[6] + terms[7]
        t8c = terms[8] + const
        y = (t01 + t23) + (t45 + t67) + t8c
        o_ref[pl.ds(r, _CHUNK_R), pl.ds(c, _CHUNK_C)] = y
        return carry

    jax.lax.fori_loop(0, nchunks, chunk, 0)


@functools.partial(jax.jit, static_argnames=())
def kernel(x, knot_ys):
    orig_shape = x.shape
    rows = x.shape[0] * x.shape[1]
    cols = x.shape[2]
    x2 = x.reshape(rows, cols)
    ys2 = knot_ys.reshape(1, _NUM_KNOTS)

    block_rows = 512
    grid = (rows // block_rows,)
    out = pl.pallas_call(
        _spline_body,
        grid=grid,
        in_specs=[
            pl.BlockSpec((1, _NUM_KNOTS), lambda i: (0, 0)),
            pl.BlockSpec((block_rows, cols), lambda i: (i, 0)),
        ],
        out_specs=pl.BlockSpec((block_rows, cols), lambda i: (i, 0)),
        out_shape=jax.ShapeDtypeStruct((rows, cols), x.dtype),
    )(ys2, x2)
    return out.reshape(orig_shape)
